# Initial kernel scaffold; baseline (speedup 1.0000x reference)
import jax
import jax.numpy as jnp
from jax.experimental import pallas as pl


def kernel(users, adjacent_items, intermediate_items, distant_items, graph_rows, graph_cols, graph_vals, theta_user, theta_item, w1_user, w2_user, w1_item, w2_item):
    raise NotImplementedError("write your pallas kernel here")


# speedup vs baseline: 6.5491x; 6.5491x over previous
import functools

import jax
import jax.numpy as jnp
from jax import lax
from jax.experimental import pallas as pl
from jax.experimental.pallas import tpu as pltpu
from jax.experimental.pallas import tpu_sc as plsc

NUM_USERS = 25000
NUM_ITEMS = 25000
N = NUM_USERS + NUM_ITEMS
C = 64
CH = 32
E = 800000
B = 4096
N_LAYERS = 2

NTILES = 16
NSC = 2
EPT = E // NTILES
RPT = 3200
RPT_LAST = N - 15 * RPT
CHUNK = 80
NCH = EPT // CHUNK

_SC_MESH = plsc.VectorSubcoreMesh(
    core_axis_name="c", subcore_axis_name="s", num_cores=NSC,
    num_subcores=NTILES)


_A_ROWS = 2000
_A_NB = N // _A_ROWS


def _softmax_body(x_ref, o_ref):
    j = pl.program_id(0)
    x = x_ref[...]
    m = jnp.max(x, axis=-1, keepdims=True)
    e = jnp.exp(x - m)
    y = e / jnp.sum(e, axis=-1, keepdims=True)
    o_ref[...] = jnp.where(j == 0, y[:, :CH], y[:, CH:])


def _softmax_stacked(theta_cat):
    return pl.pallas_call(
        _softmax_body,
        grid=(2, _A_NB),
        in_specs=[pl.BlockSpec((_A_ROWS, C), lambda j, i: (i, 0))],
        out_specs=pl.BlockSpec((_A_ROWS, CH), lambda j, i: (j * _A_NB + i, 0)),
        out_shape=jax.ShapeDtypeStruct((2 * N, CH), jnp.float32),
    )(theta_cat)


def _prop_body(rows_h, cols_h, vals_h, th0_h, zer_h, th1_h, th2_h,
               acc,
               cv0, cv1, rv0, rv1, vv0, vv1, gv0, gv1,
               isem0, isem1, gsem0, gsem1):
    c = lax.axis_index("c")
    s = lax.axis_index("s")
    ebase = s * EPT
    rowoff = c * N

    bufs = ((cv0, rv0, vv0, gv0, isem0, gsem0),
            (cv1, rv1, vv1, gv1, isem1, gsem1))

    def issue_idx(j, b):
        cv, rv, vv, _, isem, _ = bufs[b]
        off = ebase + j * CHUNK
        pltpu.async_copy(cols_h.at[pl.ds(off, CHUNK)], cv, isem)
        pltpu.async_copy(rows_h.at[pl.ds(off, CHUNK)], rv, isem)
        pltpu.async_copy(vals_h.at[pl.ds(off, CHUNK)], vv, isem)

    def wait_idx(b):
        cv, rv, vv, _, isem, _ = bufs[b]
        pltpu.make_async_copy(cols_h.at[pl.ds(0, CHUNK)], cv, isem).wait()
        pltpu.make_async_copy(rows_h.at[pl.ds(0, CHUNK)], rv, isem).wait()
        pltpu.make_async_copy(vals_h.at[pl.ds(0, CHUNK)], vv, isem).wait()

    def run_layer(src_h, dst_h):
        @pl.when(s < 15)
        def _z_main():
            pltpu.sync_copy(zer_h, acc.at[pl.ds(s * RPT, RPT)])

        @pl.when(s == 15)
        def _z_last():
            pltpu.sync_copy(zer_h.at[pl.ds(0, RPT_LAST)],
                            acc.at[pl.ds(15 * RPT, RPT_LAST)])

        plsc.subcore_barrier()

        def issue_gather(b):
            cv, _, _, gv, _, gsem = bufs[b]
            for i in range(CHUNK // 16):
                cv[pl.ds(i * 16, 16)] = cv[pl.ds(i * 16, 16)] + rowoff
            pltpu.async_copy(src_h.at[cv], gv, gsem)

        def wait_gather(b):
            cv, _, _, gv, _, gsem = bufs[b]
            pltpu.make_async_copy(src_h.at[cv], gv, gsem).wait()

        def scale_scatter(b):
            _, rv, vv, gv, _, _ = bufs[b]
            for g in range(CHUNK // 16):
                valv = vv[pl.ds(g * 16, 16)]
                for l in range(16):
                    e = g * 16 + l
                    vb = jnp.full((16,), valv[l], jnp.float32)
                    gv[e, pl.ds(0, 16)] = gv[e, pl.ds(0, 16)] * vb
                    gv[e, pl.ds(16, 16)] = gv[e, pl.ds(16, 16)] * vb
            pltpu.sync_copy(gv, acc.at[rv], add=True)

        issue_idx(0, 0)
        issue_idx(1, 1)

        @pl.loop(0, NCH - 1, step=2)
        def _pair(t):
            for b in (0, 1):
                j = t + b
                wait_idx(b)
                issue_gather(b)
                ob = 1 - b
                if b == 0:
                    @pl.when(t > 0)
                    def _prev():
                        wait_gather(ob)
                        scale_scatter(ob)
                        issue_idx(j + 1, ob)
                else:
                    wait_gather(ob)
                    scale_scatter(ob)
                    issue_idx(j + 1, ob)

        wait_idx(0)
        issue_gather(0)
        wait_gather(1)
        scale_scatter(1)
        wait_gather(0)
        scale_scatter(0)

        plsc.subcore_barrier()

        @pl.when(s < 15)
        def _w_main():
            pltpu.sync_copy(acc.at[pl.ds(s * RPT, RPT)],
                            dst_h.at[pl.ds(rowoff + s * RPT, RPT)])

        @pl.when(s == 15)
        def _w_last():
            pltpu.sync_copy(acc.at[pl.ds(15 * RPT, RPT_LAST)],
                            dst_h.at[pl.ds(rowoff + 15 * RPT, RPT_LAST)])

        plsc.subcore_barrier()

    run_layer(th0_h, th1_h)
    run_layer(th1_h, th2_h)


def _propagate(graph_rows, graph_cols, graph_vals, th0, zer):
    f = pl.kernel(
        _prop_body,
        out_type=(jax.ShapeDtypeStruct((2 * N, CH), jnp.float32),
                  jax.ShapeDtypeStruct((2 * N, CH), jnp.float32)),
        mesh=_SC_MESH,
        scratch_types=(
            pltpu.VMEM_SHARED((N, CH), jnp.float32),
            pltpu.VMEM((CHUNK,), jnp.int32), pltpu.VMEM((CHUNK,), jnp.int32),
            pltpu.VMEM((CHUNK,), jnp.int32), pltpu.VMEM((CHUNK,), jnp.int32),
            pltpu.VMEM((CHUNK,), jnp.float32), pltpu.VMEM((CHUNK,), jnp.float32),
            pltpu.VMEM((CHUNK, CH), jnp.float32), pltpu.VMEM((CHUNK, CH), jnp.float32),
            pltpu.SemaphoreType.DMA, pltpu.SemaphoreType.DMA,
            pltpu.SemaphoreType.DMA, pltpu.SemaphoreType.DMA,
        ),
        compiler_params=pltpu.CompilerParams(use_tc_tiling_on_sc=False),
    )
    return f(graph_rows, graph_cols, graph_vals, th0, zer)


NW = NSC * NTILES
BPW = B // NW
BCHUNK = 16
NPB = 7
GR = BCHUNK * NPB


def _sigmoid(x):
    return 1.0 / (1.0 + jnp.exp(-x))


def _batch_body(idx7_h, w1_h, w2_h, th0_h, th1_h, th2_h,
                o_adj_h, o_int_h, o_dis_h, o_sadj_h, o_sint_h, o_sdis_h,
                idx_v, idxo_v, g0l, g0h, g1l, g1h, g2l, g2h, w1g, w2g,
                b_adj, b_int, b_dis, b_sadj, b_sint, b_sdis, gsem):
    c = lax.axis_index("c")
    s = lax.axis_index("s")
    w = s * NSC + c
    base_b = w * BPW

    obufs = (b_adj, b_int, b_dis, b_sadj, b_sint, b_sdis)

    @pl.loop(0, BPW // BCHUNK)
    def _chunk(t):
        ib = (base_b + t * BCHUNK) * NPB
        pltpu.sync_copy(idx7_h.at[pl.ds(ib, GR)], idx_v)
        for i in range(GR // 16):
            idxo_v[pl.ds(i * 16, 16)] = idx_v[pl.ds(i * 16, 16)] + N
        cps = (
            pltpu.async_copy(th0_h.at[idx_v], g0l, gsem),
            pltpu.async_copy(th0_h.at[idxo_v], g0h, gsem),
            pltpu.async_copy(th1_h.at[idx_v], g1l, gsem),
            pltpu.async_copy(th1_h.at[idxo_v], g1h, gsem),
            pltpu.async_copy(th2_h.at[idx_v], g2l, gsem),
            pltpu.async_copy(th2_h.at[idxo_v], g2h, gsem),
            pltpu.async_copy(w1_h.at[idx_v], w1g, gsem),
            pltpu.async_copy(w2_h.at[idx_v], w2g, gsem),
        )
        for cp in cps:
            cp.wait()

        @pl.loop(0, BCHUNK // 16)
        def _group(g):
            lanes = lax.iota(jnp.int32, 16)
            rowb = (g * 16 + lanes) * NPB
            rows = [rowb + p for p in range(NPB)]
            w1v = [plsc.load_gather(w1g, [rows[p]]) for p in range(NPB)]
            w2v = [plsc.load_gather(w2g, [rows[p]]) for p in range(NPB)]

            def half_accum(garrs, f, accs):
                colv = jnp.bitwise_and(jnp.full((16,), f, jnp.int32) + lanes,
                                       CH - 1)
                x0 = [plsc.load_gather(garrs[0], [rows[p], colv])
                      for p in range(NPB)]
                x1 = [plsc.load_gather(garrs[1], [rows[p], colv])
                      for p in range(NPB)]
                x2 = [plsc.load_gather(garrs[2], [rows[p], colv])
                      for p in range(NPB)]
                z1 = [_sigmoid(x1[p] * w1v[p] + w2v[p]) for p in range(NPB)]
                z2 = [_sigmoid(x2[p] * w1v[p] + w2v[p]) for p in range(NPB)]
                z0 = x0
                lgc = [(x0[p] + x1[p] + x2[p]) * (1.0 / 3.0)
                       for p in range(NPB)]
                lgc_mi = (lgc[2] + lgc[3]) * 0.5
                lgc_md = (lgc[4] + lgc[5] + lgc[6]) * (1.0 / 3.0)
                zmi = [(zk[2] + zk[3]) * 0.5 for zk in (z0, z1, z2)]
                zmd = [(zk[4] + zk[5] + zk[6]) * (1.0 / 3.0)
                       for zk in (z0, z1, z2)]
                a_adj, a_int, a_dis, a_sadj, a_sint, a_sdis = accs
                a_adj = a_adj + lgc[0] * lgc[1]
                a_int = a_int + lgc[0] * lgc_mi
                a_dis = a_dis + lgc[0] * lgc_md
                a_sadj = a_sadj + (z0[0] * z1[1] + z1[0] * z0[1]
                                   + z1[0] * z2[1] + z2[0] * z1[1])
                a_sint = a_sint + (z0[0] * zmi[1] + z1[0] * zmi[0]
                                   + z1[0] * zmi[2] + z2[0] * zmi[1])
                a_sdis = a_sdis + (z0[0] * zmd[1] + z1[0] * zmd[0]
                                   + z1[0] * zmd[2] + z2[0] * zmd[1])
                return (a_adj, a_int, a_dis, a_sadj, a_sint, a_sdis)

            zero = jnp.zeros((16,), jnp.float32)
            accs = (zero,) * 6
            accs = lax.fori_loop(
                0, CH, lambda f, a: half_accum((g0l, g1l, g2l), f, a), accs)
            accs = lax.fori_loop(
                0, CH, lambda f, a: half_accum((g0h, g1h, g2h), f, a), accs)
            a_adj, a_int, a_dis, a_sadj, a_sint, a_sdis = accs
            off = t * BCHUNK + g * 16
            b_adj[pl.ds(off, 16)] = _sigmoid(a_adj)
            b_int[pl.ds(off, 16)] = _sigmoid(a_int)
            b_dis[pl.ds(off, 16)] = _sigmoid(a_dis)
            b_sadj[pl.ds(off, 16)] = a_sadj * 0.25
            b_sint[pl.ds(off, 16)] = a_sint * 0.25
            b_sdis[pl.ds(off, 16)] = a_sdis * 0.25

    outs = (o_adj_h, o_int_h, o_dis_h, o_sadj_h, o_sint_h, o_sdis_h)
    for ob, oh in zip(obufs, outs):
        pltpu.sync_copy(ob, oh.at[pl.ds(base_b, BPW)])


def _batch_phase(idx7, w1, w2, th0, th1, th2):
    out1 = jax.ShapeDtypeStruct((B,), jnp.float32)
    f = pl.kernel(
        _batch_body,
        out_type=(out1,) * 6,
        mesh=_SC_MESH,
        scratch_types=(
            pltpu.VMEM((GR,), jnp.int32), pltpu.VMEM((GR,), jnp.int32),
            pltpu.VMEM((GR, CH), jnp.float32), pltpu.VMEM((GR, CH), jnp.float32),
            pltpu.VMEM((GR, CH), jnp.float32), pltpu.VMEM((GR, CH), jnp.float32),
            pltpu.VMEM((GR, CH), jnp.float32), pltpu.VMEM((GR, CH), jnp.float32),
            pltpu.VMEM((GR,), jnp.float32), pltpu.VMEM((GR,), jnp.float32),
            pltpu.VMEM((BPW,), jnp.float32), pltpu.VMEM((BPW,), jnp.float32),
            pltpu.VMEM((BPW,), jnp.float32), pltpu.VMEM((BPW,), jnp.float32),
            pltpu.VMEM((BPW,), jnp.float32), pltpu.VMEM((BPW,), jnp.float32),
            pltpu.SemaphoreType.DMA,
        ),
        compiler_params=pltpu.CompilerParams(
            use_tc_tiling_on_sc=False, needs_layout_passes=False),
    )
    return f(idx7, w1, w2, th0, th1, th2)


def kernel(users, adjacent_items, intermediate_items, distant_items,
           graph_rows, graph_cols, graph_vals,
           theta_user, theta_item, w1_user, w2_user, w1_item, w2_item):
    theta_cat = jnp.concatenate([theta_user, theta_item], axis=0)
    th0 = _softmax_stacked(theta_cat)

    zer = jnp.zeros((RPT, CH), jnp.float32)
    rows32 = graph_rows.astype(jnp.int32)
    cols32 = graph_cols.astype(jnp.int32)
    th1, th2 = _propagate(rows32, cols32, graph_vals, th0, zer)

    items = jnp.concatenate(
        [adjacent_items[:, None], intermediate_items, distant_items], axis=1)
    idx7 = jnp.concatenate(
        [users[:, None], items + NUM_USERS], axis=1).astype(jnp.int32).reshape(-1)
    w1 = jnp.concatenate([w1_user[:, 0], w1_item[:, 0]])
    w2 = jnp.concatenate([w2_user[:, 0], w2_item[:, 0]])

    outs = _batch_phase(idx7, w1, w2, th0, th1, th2)
    return jnp.stack(outs, axis=0)


# speedup vs baseline: 9.1837x; 1.4023x over previous
import functools

import jax
import jax.numpy as jnp
from jax import lax
from jax.experimental import pallas as pl
from jax.experimental.pallas import tpu as pltpu
from jax.experimental.pallas import tpu_sc as plsc

NUM_USERS = 25000
NUM_ITEMS = 25000
N = NUM_USERS + NUM_ITEMS
C = 64
CH = 32
E = 800000
B = 4096
N_LAYERS = 2

NTILES = 16
NSC = 2
EPT = E // NTILES
RPT = 3200
RPT_LAST = N - 15 * RPT
CHUNK = 128
NCH = EPT // CHUNK
TAIL = EPT - NCH * CHUNK

_SC_MESH = plsc.VectorSubcoreMesh(
    core_axis_name="c", subcore_axis_name="s", num_cores=NSC,
    num_subcores=NTILES)


_A_ROWS = 2000
_A_NB = N // _A_ROWS


def _softmax_body(x_ref, o_ref):
    j = pl.program_id(0)
    x = x_ref[...]
    m = jnp.max(x, axis=-1, keepdims=True)
    e = jnp.exp(x - m)
    y = e / jnp.sum(e, axis=-1, keepdims=True)
    o_ref[...] = jnp.where(j == 0, y[:, :CH], y[:, CH:])


def _softmax_stacked(theta_cat):
    return pl.pallas_call(
        _softmax_body,
        grid=(2, _A_NB),
        in_specs=[pl.BlockSpec((_A_ROWS, C), lambda j, i: (i, 0))],
        out_specs=pl.BlockSpec((_A_ROWS, CH), lambda j, i: (j * _A_NB + i, 0)),
        out_shape=jax.ShapeDtypeStruct((2 * N, CH), jnp.float32),
    )(theta_cat)


def _prop_body(rows_h, cols_h, vals_h, th0_h, zer_h, th1_h, th2_h,
               acc,
               cv0, cv1, rv0, rv1, vv0, vv1, gv0, gv1, rs0, rs1,
               cvt, rvt, vvt, gvt,
               isem0, isem1, gsem0, gsem1, ssem0, ssem1):
    c = lax.axis_index("c")
    s = lax.axis_index("s")
    ebase = s * EPT
    rowoff = c * N

    bufs = ((cv0, rv0, vv0, gv0, rs0, isem0, gsem0, ssem0),
            (cv1, rv1, vv1, gv1, rs1, isem1, gsem1, ssem1))

    def issue_idx(j, b):
        cv, rv, vv, _, _, isem, _, _ = bufs[b]
        off = ebase + j * CHUNK
        pltpu.async_copy(cols_h.at[pl.ds(off, CHUNK)], cv, isem)
        pltpu.async_copy(rows_h.at[pl.ds(off, CHUNK)], rv, isem)
        pltpu.async_copy(vals_h.at[pl.ds(off, CHUNK)], vv, isem)

    def wait_idx(b):
        cv, rv, vv, _, _, isem, _, _ = bufs[b]
        pltpu.make_async_copy(cols_h.at[pl.ds(0, CHUNK)], cv, isem).wait()
        pltpu.make_async_copy(rows_h.at[pl.ds(0, CHUNK)], rv, isem).wait()
        pltpu.make_async_copy(vals_h.at[pl.ds(0, CHUNK)], vv, isem).wait()

    def run_layer(src_h, dst_h):
        @pl.when(s < 15)
        def _z_main():
            pltpu.sync_copy(zer_h, acc.at[pl.ds(s * RPT, RPT)])

        @pl.when(s == 15)
        def _z_last():
            pltpu.sync_copy(zer_h.at[pl.ds(0, RPT_LAST)],
                            acc.at[pl.ds(15 * RPT, RPT_LAST)])

        plsc.subcore_barrier()

        def issue_gather(b):
            cv, _, _, gv, _, _, gsem, _ = bufs[b]
            for i in range(CHUNK // 16):
                cv[pl.ds(i * 16, 16)] = cv[pl.ds(i * 16, 16)] + rowoff
            pltpu.async_copy(src_h.at[cv], gv, gsem)

        def wait_gather(b):
            cv, _, _, gv, _, _, gsem, _ = bufs[b]
            pltpu.make_async_copy(src_h.at[cv], gv, gsem).wait()

        def scale_scatter(b):
            _, rv, vv, gv, rs, _, _, ssem = bufs[b]
            for g in range(CHUNK // 16):
                valv = vv[pl.ds(g * 16, 16)]
                for l in range(16):
                    e = g * 16 + l
                    vb = jnp.full((16,), valv[l], jnp.float32)
                    gv[e, pl.ds(0, 16)] = gv[e, pl.ds(0, 16)] * vb
                    gv[e, pl.ds(16, 16)] = gv[e, pl.ds(16, 16)] * vb
            for i in range(CHUNK // 16):
                rs[pl.ds(i * 16, 16)] = rv[pl.ds(i * 16, 16)]
            pltpu.async_copy(gv, acc.at[rs], ssem, add=True)

        def wait_scatter(b):
            _, _, _, gv, rs, _, _, ssem = bufs[b]
            pltpu.make_async_copy(gv, acc.at[rs], ssem).wait()

        issue_idx(0, 0)
        issue_idx(1, 1)

        @pl.loop(0, NCH - 2, step=2)
        def _pair(t):
            for b in (0, 1):
                j = t + b
                wait_idx(b)

                @pl.when(t > 0)
                def _ws():
                    wait_scatter(b)

                issue_gather(b)
                ob = 1 - b
                if b == 0:
                    @pl.when(t > 0)
                    def _prev():
                        wait_gather(ob)
                        scale_scatter(ob)
                        issue_idx(j + 1, ob)
                else:
                    wait_gather(ob)
                    scale_scatter(ob)
                    issue_idx(j + 1, ob)

        toff = ebase + NCH * CHUNK
        wait_idx(0)
        wait_scatter(0)
        issue_gather(0)
        pltpu.async_copy(cols_h.at[pl.ds(toff, TAIL)], cvt, isem0)
        pltpu.async_copy(rows_h.at[pl.ds(toff, TAIL)], rvt, isem0)
        pltpu.async_copy(vals_h.at[pl.ds(toff, TAIL)], vvt, isem0)
        wait_gather(1)
        scale_scatter(1)
        issue_idx(NCH - 1, 1)
        wait_idx(1)
        wait_scatter(1)
        issue_gather(1)
        wait_gather(0)
        scale_scatter(0)
        wait_gather(1)
        scale_scatter(1)
        pltpu.make_async_copy(cols_h.at[pl.ds(0, TAIL)], cvt, isem0).wait()
        pltpu.make_async_copy(rows_h.at[pl.ds(0, TAIL)], rvt, isem0).wait()
        pltpu.make_async_copy(vals_h.at[pl.ds(0, TAIL)], vvt, isem0).wait()
        for i in range(TAIL // 16):
            cvt[pl.ds(i * 16, 16)] = cvt[pl.ds(i * 16, 16)] + rowoff
        pltpu.async_copy(src_h.at[cvt], gvt, gsem0)
        pltpu.make_async_copy(src_h.at[cvt], gvt, gsem0).wait()
        for g in range(TAIL // 16):
            valv = vvt[pl.ds(g * 16, 16)]
            for l in range(16):
                e = g * 16 + l
                vb = jnp.full((16,), valv[l], jnp.float32)
                gvt[e, pl.ds(0, 16)] = gvt[e, pl.ds(0, 16)] * vb
                gvt[e, pl.ds(16, 16)] = gvt[e, pl.ds(16, 16)] * vb
        pltpu.sync_copy(gvt, acc.at[rvt], add=True)
        wait_scatter(0)
        wait_scatter(1)

        plsc.subcore_barrier()

        @pl.when(s < 15)
        def _w_main():
            pltpu.sync_copy(acc.at[pl.ds(s * RPT, RPT)],
                            dst_h.at[pl.ds(rowoff + s * RPT, RPT)])

        @pl.when(s == 15)
        def _w_last():
            pltpu.sync_copy(acc.at[pl.ds(15 * RPT, RPT_LAST)],
                            dst_h.at[pl.ds(rowoff + 15 * RPT, RPT_LAST)])

        plsc.subcore_barrier()

    run_layer(th0_h, th1_h)
    run_layer(th1_h, th2_h)


def _propagate(graph_rows, graph_cols, graph_vals, th0, zer):
    f = pl.kernel(
        _prop_body,
        out_type=(jax.ShapeDtypeStruct((2 * N, CH), jnp.float32),
                  jax.ShapeDtypeStruct((2 * N, CH), jnp.float32)),
        mesh=_SC_MESH,
        scratch_types=(
            pltpu.VMEM_SHARED((N, CH), jnp.float32),
            pltpu.VMEM((CHUNK,), jnp.int32), pltpu.VMEM((CHUNK,), jnp.int32),
            pltpu.VMEM((CHUNK,), jnp.int32), pltpu.VMEM((CHUNK,), jnp.int32),
            pltpu.VMEM((CHUNK,), jnp.float32), pltpu.VMEM((CHUNK,), jnp.float32),
            pltpu.VMEM((CHUNK, CH), jnp.float32), pltpu.VMEM((CHUNK, CH), jnp.float32),
            pltpu.VMEM((CHUNK,), jnp.int32), pltpu.VMEM((CHUNK,), jnp.int32),
            pltpu.VMEM((TAIL,), jnp.int32), pltpu.VMEM((TAIL,), jnp.int32),
            pltpu.VMEM((TAIL,), jnp.float32), pltpu.VMEM((TAIL, CH), jnp.float32),
            pltpu.SemaphoreType.DMA, pltpu.SemaphoreType.DMA,
            pltpu.SemaphoreType.DMA, pltpu.SemaphoreType.DMA,
            pltpu.SemaphoreType.DMA, pltpu.SemaphoreType.DMA,
        ),
        compiler_params=pltpu.CompilerParams(use_tc_tiling_on_sc=False),
    )
    return f(graph_rows, graph_cols, graph_vals, th0, zer)


NW = NSC * NTILES
BPW = B // NW
BCHUNK = 16
NPB = 7
GR = BCHUNK * NPB


def _sigmoid(x):
    return 1.0 / (1.0 + jnp.exp(-x))


def _batch_body(idx7_h, w1_h, w2_h, th0_h, th1_h, th2_h,
                o_adj_h, o_int_h, o_dis_h, o_sadj_h, o_sint_h, o_sdis_h,
                idx_v, idxo_v, g0l, g0h, g1l, g1h, g2l, g2h, w1g, w2g,
                b_adj, b_int, b_dis, b_sadj, b_sint, b_sdis, gsem):
    c = lax.axis_index("c")
    s = lax.axis_index("s")
    w = s * NSC + c
    base_b = w * BPW

    obufs = (b_adj, b_int, b_dis, b_sadj, b_sint, b_sdis)

    @pl.loop(0, BPW // BCHUNK)
    def _chunk(t):
        ib = (base_b + t * BCHUNK) * NPB
        pltpu.sync_copy(idx7_h.at[pl.ds(ib, GR)], idx_v)
        for i in range(GR // 16):
            idxo_v[pl.ds(i * 16, 16)] = idx_v[pl.ds(i * 16, 16)] + N
        cps = (
            pltpu.async_copy(th0_h.at[idx_v], g0l, gsem),
            pltpu.async_copy(th0_h.at[idxo_v], g0h, gsem),
            pltpu.async_copy(th1_h.at[idx_v], g1l, gsem),
            pltpu.async_copy(th1_h.at[idxo_v], g1h, gsem),
            pltpu.async_copy(th2_h.at[idx_v], g2l, gsem),
            pltpu.async_copy(th2_h.at[idxo_v], g2h, gsem),
            pltpu.async_copy(w1_h.at[idx_v], w1g, gsem),
            pltpu.async_copy(w2_h.at[idx_v], w2g, gsem),
        )
        for cp in cps:
            cp.wait()

        @pl.loop(0, BCHUNK // 16)
        def _group(g):
            lanes = lax.iota(jnp.int32, 16)
            rowb = (g * 16 + lanes) * NPB
            rows = [rowb + p for p in range(NPB)]
            w1v = [plsc.load_gather(w1g, [rows[p]]) for p in range(NPB)]
            w2v = [plsc.load_gather(w2g, [rows[p]]) for p in range(NPB)]

            def half_accum(garrs, f, accs):
                colv = jnp.bitwise_and(jnp.full((16,), f, jnp.int32) + lanes,
                                       CH - 1)
                x0 = [plsc.load_gather(garrs[0], [rows[p], colv])
                      for p in range(NPB)]
                x1 = [plsc.load_gather(garrs[1], [rows[p], colv])
                      for p in range(NPB)]
                x2 = [plsc.load_gather(garrs[2], [rows[p], colv])
                      for p in range(NPB)]
                z1 = [_sigmoid(x1[p] * w1v[p] + w2v[p]) for p in range(NPB)]
                z2 = [_sigmoid(x2[p] * w1v[p] + w2v[p]) for p in range(NPB)]
                z0 = x0
                lgc = [(x0[p] + x1[p] + x2[p]) * (1.0 / 3.0)
                       for p in range(NPB)]
                lgc_mi = (lgc[2] + lgc[3]) * 0.5
                lgc_md = (lgc[4] + lgc[5] + lgc[6]) * (1.0 / 3.0)
                zmi = [(zk[2] + zk[3]) * 0.5 for zk in (z0, z1, z2)]
                zmd = [(zk[4] + zk[5] + zk[6]) * (1.0 / 3.0)
                       for zk in (z0, z1, z2)]
                a_adj, a_int, a_dis, a_sadj, a_sint, a_sdis = accs
                a_adj = a_adj + lgc[0] * lgc[1]
                a_int = a_int + lgc[0] * lgc_mi
                a_dis = a_dis + lgc[0] * lgc_md
                a_sadj = a_sadj + (z0[0] * z1[1] + z1[0] * z0[1]
                                   + z1[0] * z2[1] + z2[0] * z1[1])
                a_sint = a_sint + (z0[0] * zmi[1] + z1[0] * zmi[0]
                                   + z1[0] * zmi[2] + z2[0] * zmi[1])
                a_sdis = a_sdis + (z0[0] * zmd[1] + z1[0] * zmd[0]
                                   + z1[0] * zmd[2] + z2[0] * zmd[1])
                return (a_adj, a_int, a_dis, a_sadj, a_sint, a_sdis)

            zero = jnp.zeros((16,), jnp.float32)
            accs = (zero,) * 6
            accs = lax.fori_loop(
                0, CH, lambda f, a: half_accum((g0l, g1l, g2l), f, a), accs)
            accs = lax.fori_loop(
                0, CH, lambda f, a: half_accum((g0h, g1h, g2h), f, a), accs)
            a_adj, a_int, a_dis, a_sadj, a_sint, a_sdis = accs
            off = t * BCHUNK + g * 16
            b_adj[pl.ds(off, 16)] = _sigmoid(a_adj)
            b_int[pl.ds(off, 16)] = _sigmoid(a_int)
            b_dis[pl.ds(off, 16)] = _sigmoid(a_dis)
            b_sadj[pl.ds(off, 16)] = a_sadj * 0.25
            b_sint[pl.ds(off, 16)] = a_sint * 0.25
            b_sdis[pl.ds(off, 16)] = a_sdis * 0.25

    outs = (o_adj_h, o_int_h, o_dis_h, o_sadj_h, o_sint_h, o_sdis_h)
    for ob, oh in zip(obufs, outs):
        pltpu.sync_copy(ob, oh.at[pl.ds(base_b, BPW)])


def _batch_phase(idx7, w1, w2, th0, th1, th2):
    out1 = jax.ShapeDtypeStruct((B,), jnp.float32)
    f = pl.kernel(
        _batch_body,
        out_type=(out1,) * 6,
        mesh=_SC_MESH,
        scratch_types=(
            pltpu.VMEM((GR,), jnp.int32), pltpu.VMEM((GR,), jnp.int32),
            pltpu.VMEM((GR, CH), jnp.float32), pltpu.VMEM((GR, CH), jnp.float32),
            pltpu.VMEM((GR, CH), jnp.float32), pltpu.VMEM((GR, CH), jnp.float32),
            pltpu.VMEM((GR, CH), jnp.float32), pltpu.VMEM((GR, CH), jnp.float32),
            pltpu.VMEM((GR,), jnp.float32), pltpu.VMEM((GR,), jnp.float32),
            pltpu.VMEM((BPW,), jnp.float32), pltpu.VMEM((BPW,), jnp.float32),
            pltpu.VMEM((BPW,), jnp.float32), pltpu.VMEM((BPW,), jnp.float32),
            pltpu.VMEM((BPW,), jnp.float32), pltpu.VMEM((BPW,), jnp.float32),
            pltpu.SemaphoreType.DMA,
        ),
        compiler_params=pltpu.CompilerParams(
            use_tc_tiling_on_sc=False, needs_layout_passes=False),
    )
    return f(idx7, w1, w2, th0, th1, th2)


def kernel(users, adjacent_items, intermediate_items, distant_items,
           graph_rows, graph_cols, graph_vals,
           theta_user, theta_item, w1_user, w2_user, w1_item, w2_item):
    theta_cat = jnp.concatenate([theta_user, theta_item], axis=0)
    th0 = _softmax_stacked(theta_cat)

    zer = jnp.zeros((RPT, CH), jnp.float32)
    rows32 = graph_rows.astype(jnp.int32)
    cols32 = graph_cols.astype(jnp.int32)
    th1, th2 = _propagate(rows32, cols32, graph_vals, th0, zer)

    items = jnp.concatenate(
        [adjacent_items[:, None], intermediate_items, distant_items], axis=1)
    idx7 = jnp.concatenate(
        [users[:, None], items + NUM_USERS], axis=1).astype(jnp.int32).reshape(-1)
    w1 = jnp.concatenate([w1_user[:, 0], w1_item[:, 0]])
    w2 = jnp.concatenate([w2_user[:, 0], w2_item[:, 0]])

    outs = _batch_phase(idx7, w1, w2, th0, th1, th2)
    return jnp.stack(outs, axis=0)


# speedup vs baseline: 12.8523x; 1.3995x over previous
import functools

import jax
import jax.numpy as jnp
from jax import lax
from jax.experimental import pallas as pl
from jax.experimental.pallas import tpu as pltpu
from jax.experimental.pallas import tpu_sc as plsc

NUM_USERS = 25000
NUM_ITEMS = 25000
N = NUM_USERS + NUM_ITEMS
C = 64
CH = 32
E = 800000
B = 4096
N_LAYERS = 2

NTILES = 16
NSC = 2
EPT = E // NTILES
RPT = 3200
RPT_LAST = N - 15 * RPT
CHUNK = 400
NCH = EPT // CHUNK

_SC_MESH = plsc.VectorSubcoreMesh(
    core_axis_name="c", subcore_axis_name="s", num_cores=NSC,
    num_subcores=NTILES)


_A_ROWS = 2000
_A_NB = N // _A_ROWS


def _softmax_body(x_ref, o_ref):
    j = pl.program_id(0)
    x = x_ref[...]
    m = jnp.max(x, axis=-1, keepdims=True)
    e = jnp.exp(x - m)
    y = e / jnp.sum(e, axis=-1, keepdims=True)
    o_ref[...] = jnp.where(j == 0, y[:, :CH], y[:, CH:])


def _softmax_stacked(theta_cat):
    return pl.pallas_call(
        _softmax_body,
        grid=(2, _A_NB),
        in_specs=[pl.BlockSpec((_A_ROWS, C), lambda j, i: (i, 0))],
        out_specs=pl.BlockSpec((_A_ROWS, CH), lambda j, i: (j * _A_NB + i, 0)),
        out_shape=jax.ShapeDtypeStruct((2 * N, CH), jnp.float32),
    )(theta_cat)


def _prop_body(rows_h, cols_h, vals_h, th0_h, zer_h, th1_h, th2_h,
               acc,
               cv0, cv1, rv0, rv1, vv0, vv1, gv0, gv1, rs0, rs1,
               isem0, isem1, gsem0, gsem1, ssem0, ssem1):
    c = lax.axis_index("c")
    s = lax.axis_index("s")
    ebase = s * EPT
    rowoff = c * N

    bufs = ((cv0, rv0, vv0, gv0, rs0, isem0, gsem0, ssem0),
            (cv1, rv1, vv1, gv1, rs1, isem1, gsem1, ssem1))

    def issue_idx(j, b):
        cv, rv, vv, _, _, isem, _, _ = bufs[b]
        off = ebase + j * CHUNK
        pltpu.async_copy(cols_h.at[pl.ds(off, CHUNK)], cv, isem)
        pltpu.async_copy(rows_h.at[pl.ds(off, CHUNK)], rv, isem)
        pltpu.async_copy(vals_h.at[pl.ds(off, CHUNK)], vv, isem)

    def wait_idx(b):
        cv, rv, vv, _, _, isem, _, _ = bufs[b]
        pltpu.make_async_copy(cols_h.at[pl.ds(0, CHUNK)], cv, isem).wait()
        pltpu.make_async_copy(rows_h.at[pl.ds(0, CHUNK)], rv, isem).wait()
        pltpu.make_async_copy(vals_h.at[pl.ds(0, CHUNK)], vv, isem).wait()

    def run_layer(src_h, dst_h):
        @pl.when(s < 15)
        def _z_main():
            pltpu.sync_copy(zer_h, acc.at[pl.ds(s * RPT, RPT)])

        @pl.when(s == 15)
        def _z_last():
            pltpu.sync_copy(zer_h.at[pl.ds(0, RPT_LAST)],
                            acc.at[pl.ds(15 * RPT, RPT_LAST)])

        plsc.subcore_barrier()

        def issue_gather(b):
            cv, _, _, gv, _, _, gsem, _ = bufs[b]

            @pl.loop(0, CHUNK // 16)
            def _off(i):
                cv[pl.ds(i * 16, 16)] = cv[pl.ds(i * 16, 16)] + rowoff

            pltpu.async_copy(src_h.at[cv], gv, gsem)

        def wait_gather(b):
            cv, _, _, gv, _, _, gsem, _ = bufs[b]
            pltpu.make_async_copy(src_h.at[cv], gv, gsem).wait()

        def scale_scatter(b):
            _, rv, vv, gv, rs, _, _, ssem = bufs[b]

            @pl.loop(0, CHUNK // 16)
            def _sc(g):
                valv = vv[pl.ds(g * 16, 16)]
                for l in range(16):
                    e = g * 16 + l
                    vb = jnp.full((16,), valv[l], jnp.float32)
                    gv[e, pl.ds(0, 16)] = gv[e, pl.ds(0, 16)] * vb
                    gv[e, pl.ds(16, 16)] = gv[e, pl.ds(16, 16)] * vb
                rs[pl.ds(g * 16, 16)] = rv[pl.ds(g * 16, 16)]

            pltpu.async_copy(gv, acc.at[rs], ssem, add=True)

        def wait_scatter(b):
            _, _, _, gv, rs, _, _, ssem = bufs[b]
            pltpu.make_async_copy(gv, acc.at[rs], ssem).wait()

        issue_idx(0, 0)
        issue_idx(1, 1)

        @pl.loop(0, NCH - 1, step=2)
        def _pair(t):
            for b in (0, 1):
                j = t + b
                wait_idx(b)

                @pl.when(t > 0)
                def _ws():
                    wait_scatter(b)

                issue_gather(b)
                ob = 1 - b
                if b == 0:
                    @pl.when(t > 0)
                    def _prev():
                        wait_gather(ob)
                        scale_scatter(ob)
                        issue_idx(j + 1, ob)
                else:
                    wait_gather(ob)
                    scale_scatter(ob)
                    issue_idx(j + 1, ob)

        wait_idx(0)
        wait_scatter(0)
        issue_gather(0)
        wait_gather(1)
        scale_scatter(1)
        wait_gather(0)
        scale_scatter(0)
        wait_scatter(1)
        wait_scatter(0)

        plsc.subcore_barrier()

        @pl.when(s < 15)
        def _w_main():
            pltpu.sync_copy(acc.at[pl.ds(s * RPT, RPT)],
                            dst_h.at[pl.ds(rowoff + s * RPT, RPT)])

        @pl.when(s == 15)
        def _w_last():
            pltpu.sync_copy(acc.at[pl.ds(15 * RPT, RPT_LAST)],
                            dst_h.at[pl.ds(rowoff + 15 * RPT, RPT_LAST)])

        plsc.subcore_barrier()

    run_layer(th0_h, th1_h)
    run_layer(th1_h, th2_h)


def _propagate(graph_rows, graph_cols, graph_vals, th0, zer):
    f = pl.kernel(
        _prop_body,
        out_type=(jax.ShapeDtypeStruct((2 * N, CH), jnp.float32),
                  jax.ShapeDtypeStruct((2 * N, CH), jnp.float32)),
        mesh=_SC_MESH,
        scratch_types=(
            pltpu.VMEM_SHARED((N, CH), jnp.float32),
            pltpu.VMEM((CHUNK,), jnp.int32), pltpu.VMEM((CHUNK,), jnp.int32),
            pltpu.VMEM((CHUNK,), jnp.int32), pltpu.VMEM((CHUNK,), jnp.int32),
            pltpu.VMEM((CHUNK,), jnp.float32), pltpu.VMEM((CHUNK,), jnp.float32),
            pltpu.VMEM((CHUNK, CH), jnp.float32), pltpu.VMEM((CHUNK, CH), jnp.float32),
            pltpu.VMEM((CHUNK,), jnp.int32), pltpu.VMEM((CHUNK,), jnp.int32),
            pltpu.SemaphoreType.DMA, pltpu.SemaphoreType.DMA,
            pltpu.SemaphoreType.DMA, pltpu.SemaphoreType.DMA,
            pltpu.SemaphoreType.DMA, pltpu.SemaphoreType.DMA,
        ),
        compiler_params=pltpu.CompilerParams(use_tc_tiling_on_sc=False),
    )
    return f(graph_rows, graph_cols, graph_vals, th0, zer)


NW = NSC * NTILES
BPW = B // NW
BCHUNK = 16
NPB = 7
GR = BCHUNK * NPB


def _sigmoid(x):
    return 1.0 / (1.0 + jnp.exp(-x))


def _batch_body(idx7_h, w1_h, w2_h, th0_h, th1_h, th2_h,
                o_adj_h, o_int_h, o_dis_h, o_sadj_h, o_sint_h, o_sdis_h,
                idx_v, idxo_v, g0l, g0h, g1l, g1h, g2l, g2h, w1g, w2g,
                b_adj, b_int, b_dis, b_sadj, b_sint, b_sdis, gsem):
    c = lax.axis_index("c")
    s = lax.axis_index("s")
    w = s * NSC + c
    base_b = w * BPW

    obufs = (b_adj, b_int, b_dis, b_sadj, b_sint, b_sdis)

    @pl.loop(0, BPW // BCHUNK)
    def _chunk(t):
        ib = (base_b + t * BCHUNK) * NPB
        pltpu.sync_copy(idx7_h.at[pl.ds(ib, GR)], idx_v)
        for i in range(GR // 16):
            idxo_v[pl.ds(i * 16, 16)] = idx_v[pl.ds(i * 16, 16)] + N
        cps = (
            pltpu.async_copy(th0_h.at[idx_v], g0l, gsem),
            pltpu.async_copy(th0_h.at[idxo_v], g0h, gsem),
            pltpu.async_copy(th1_h.at[idx_v], g1l, gsem),
            pltpu.async_copy(th1_h.at[idxo_v], g1h, gsem),
            pltpu.async_copy(th2_h.at[idx_v], g2l, gsem),
            pltpu.async_copy(th2_h.at[idxo_v], g2h, gsem),
            pltpu.async_copy(w1_h.at[idx_v], w1g, gsem),
            pltpu.async_copy(w2_h.at[idx_v], w2g, gsem),
        )
        for cp in cps:
            cp.wait()

        @pl.loop(0, BCHUNK // 16)
        def _group(g):
            lanes = lax.iota(jnp.int32, 16)
            rowb = (g * 16 + lanes) * NPB
            rows = [rowb + p for p in range(NPB)]
            w1v = [plsc.load_gather(w1g, [rows[p]]) for p in range(NPB)]
            w2v = [plsc.load_gather(w2g, [rows[p]]) for p in range(NPB)]

            def half_accum(garrs, f, accs):
                colv = jnp.bitwise_and(jnp.full((16,), f, jnp.int32) + lanes,
                                       CH - 1)
                x0 = [plsc.load_gather(garrs[0], [rows[p], colv])
                      for p in range(NPB)]
                x1 = [plsc.load_gather(garrs[1], [rows[p], colv])
                      for p in range(NPB)]
                x2 = [plsc.load_gather(garrs[2], [rows[p], colv])
                      for p in range(NPB)]
                z1 = [_sigmoid(x1[p] * w1v[p] + w2v[p]) for p in range(NPB)]
                z2 = [_sigmoid(x2[p] * w1v[p] + w2v[p]) for p in range(NPB)]
                z0 = x0
                lgc = [(x0[p] + x1[p] + x2[p]) * (1.0 / 3.0)
                       for p in range(NPB)]
                lgc_mi = (lgc[2] + lgc[3]) * 0.5
                lgc_md = (lgc[4] + lgc[5] + lgc[6]) * (1.0 / 3.0)
                zmi = [(zk[2] + zk[3]) * 0.5 for zk in (z0, z1, z2)]
                zmd = [(zk[4] + zk[5] + zk[6]) * (1.0 / 3.0)
                       for zk in (z0, z1, z2)]
                a_adj, a_int, a_dis, a_sadj, a_sint, a_sdis = accs
                a_adj = a_adj + lgc[0] * lgc[1]
                a_int = a_int + lgc[0] * lgc_mi
                a_dis = a_dis + lgc[0] * lgc_md
                a_sadj = a_sadj + (z0[0] * z1[1] + z1[0] * z0[1]
                                   + z1[0] * z2[1] + z2[0] * z1[1])
                a_sint = a_sint + (z0[0] * zmi[1] + z1[0] * zmi[0]
                                   + z1[0] * zmi[2] + z2[0] * zmi[1])
                a_sdis = a_sdis + (z0[0] * zmd[1] + z1[0] * zmd[0]
                                   + z1[0] * zmd[2] + z2[0] * zmd[1])
                return (a_adj, a_int, a_dis, a_sadj, a_sint, a_sdis)

            zero = jnp.zeros((16,), jnp.float32)
            accs = (zero,) * 6
            accs = lax.fori_loop(
                0, CH, lambda f, a: half_accum((g0l, g1l, g2l), f, a), accs)
            accs = lax.fori_loop(
                0, CH, lambda f, a: half_accum((g0h, g1h, g2h), f, a), accs)
            a_adj, a_int, a_dis, a_sadj, a_sint, a_sdis = accs
            off = t * BCHUNK + g * 16
            b_adj[pl.ds(off, 16)] = _sigmoid(a_adj)
            b_int[pl.ds(off, 16)] = _sigmoid(a_int)
            b_dis[pl.ds(off, 16)] = _sigmoid(a_dis)
            b_sadj[pl.ds(off, 16)] = a_sadj * 0.25
            b_sint[pl.ds(off, 16)] = a_sint * 0.25
            b_sdis[pl.ds(off, 16)] = a_sdis * 0.25

    outs = (o_adj_h, o_int_h, o_dis_h, o_sadj_h, o_sint_h, o_sdis_h)
    for ob, oh in zip(obufs, outs):
        pltpu.sync_copy(ob, oh.at[pl.ds(base_b, BPW)])


def _batch_phase(idx7, w1, w2, th0, th1, th2):
    out1 = jax.ShapeDtypeStruct((B,), jnp.float32)
    f = pl.kernel(
        _batch_body,
        out_type=(out1,) * 6,
        mesh=_SC_MESH,
        scratch_types=(
            pltpu.VMEM((GR,), jnp.int32), pltpu.VMEM((GR,), jnp.int32),
            pltpu.VMEM((GR, CH), jnp.float32), pltpu.VMEM((GR, CH), jnp.float32),
            pltpu.VMEM((GR, CH), jnp.float32), pltpu.VMEM((GR, CH), jnp.float32),
            pltpu.VMEM((GR, CH), jnp.float32), pltpu.VMEM((GR, CH), jnp.float32),
            pltpu.VMEM((GR,), jnp.float32), pltpu.VMEM((GR,), jnp.float32),
            pltpu.VMEM((BPW,), jnp.float32), pltpu.VMEM((BPW,), jnp.float32),
            pltpu.VMEM((BPW,), jnp.float32), pltpu.VMEM((BPW,), jnp.float32),
            pltpu.VMEM((BPW,), jnp.float32), pltpu.VMEM((BPW,), jnp.float32),
            pltpu.SemaphoreType.DMA,
        ),
        compiler_params=pltpu.CompilerParams(
            use_tc_tiling_on_sc=False, needs_layout_passes=False),
    )
    return f(idx7, w1, w2, th0, th1, th2)


def kernel(users, adjacent_items, intermediate_items, distant_items,
           graph_rows, graph_cols, graph_vals,
           theta_user, theta_item, w1_user, w2_user, w1_item, w2_item):
    theta_cat = jnp.concatenate([theta_user, theta_item], axis=0)
    th0 = _softmax_stacked(theta_cat)

    zer = jnp.zeros((RPT, CH), jnp.float32)
    rows32 = graph_rows.astype(jnp.int32)
    cols32 = graph_cols.astype(jnp.int32)
    th1, th2 = _propagate(rows32, cols32, graph_vals, th0, zer)

    items = jnp.concatenate(
        [adjacent_items[:, None], intermediate_items, distant_items], axis=1)
    idx7 = jnp.concatenate(
        [users[:, None], items + NUM_USERS], axis=1).astype(jnp.int32).reshape(-1)
    w1 = jnp.concatenate([w1_user[:, 0], w1_item[:, 0]])
    w2 = jnp.concatenate([w2_user[:, 0], w2_item[:, 0]])

    outs = _batch_phase(idx7, w1, w2, th0, th1, th2)
    return jnp.stack(outs, axis=0)


# speedup vs baseline: 13.1110x; 1.0201x over previous
import functools

import jax
import jax.numpy as jnp
from jax import lax
from jax.experimental import pallas as pl
from jax.experimental.pallas import tpu as pltpu
from jax.experimental.pallas import tpu_sc as plsc

NUM_USERS = 25000
NUM_ITEMS = 25000
N = NUM_USERS + NUM_ITEMS
C = 64
CH = 32
E = 800000
B = 4096
N_LAYERS = 2

NTILES = 16
NSC = 2
EPT = E // NTILES
RPT = 3200
RPT_LAST = N - 15 * RPT
CHUNK = 400
NCH = EPT // CHUNK

_SC_MESH = plsc.VectorSubcoreMesh(
    core_axis_name="c", subcore_axis_name="s", num_cores=NSC,
    num_subcores=NTILES)


_A_ROWS = 2000
_A_NB = N // _A_ROWS


def _softmax_body(x_ref, o_ref):
    j = pl.program_id(0)
    x = x_ref[...]
    m = jnp.max(x, axis=-1, keepdims=True)
    e = jnp.exp(x - m)
    y = e / jnp.sum(e, axis=-1, keepdims=True)
    o_ref[...] = jnp.where(j == 0, y[:, :CH], y[:, CH:])


def _softmax_stacked(theta_cat):
    return pl.pallas_call(
        _softmax_body,
        grid=(2, _A_NB),
        in_specs=[pl.BlockSpec((_A_ROWS, C), lambda j, i: (i, 0))],
        out_specs=pl.BlockSpec((_A_ROWS, CH), lambda j, i: (j * _A_NB + i, 0)),
        out_shape=jax.ShapeDtypeStruct((2 * N, CH), jnp.float32),
    )(theta_cat)


def _prop_body(rows_h, cols_h, vals_h, th0_h, zer_h, th1_h, th2_h,
               acc,
               cv0, cv1, rv0, rv1, vv0, vv1, gv0, gv1, rs0, rs1,
               isem0, isem1, gsem0, gsem1, ssem0, ssem1):
    c = lax.axis_index("c")
    s = lax.axis_index("s")
    ebase = s * EPT
    rowoff = c * N

    bufs = ((cv0, rv0, vv0, gv0, rs0, isem0, gsem0, ssem0),
            (cv1, rv1, vv1, gv1, rs1, isem1, gsem1, ssem1))

    def issue_idx(j, b):
        cv, rv, vv, _, _, isem, _, _ = bufs[b]
        off = ebase + j * CHUNK
        pltpu.async_copy(cols_h.at[pl.ds(c * E + off, CHUNK)], cv, isem)
        pltpu.async_copy(rows_h.at[pl.ds(off, CHUNK)], rv, isem)
        pltpu.async_copy(vals_h.at[pl.ds(off, CHUNK)], vv, isem)

    def wait_idx(b):
        cv, rv, vv, _, _, isem, _, _ = bufs[b]
        pltpu.make_async_copy(cols_h.at[pl.ds(0, CHUNK)], cv, isem).wait()
        pltpu.make_async_copy(rows_h.at[pl.ds(0, CHUNK)], rv, isem).wait()
        pltpu.make_async_copy(vals_h.at[pl.ds(0, CHUNK)], vv, isem).wait()

    def run_layer(src_h, dst_h):
        @pl.when(s < 15)
        def _z_main():
            pltpu.sync_copy(zer_h, acc.at[pl.ds(s * RPT, RPT)])

        @pl.when(s == 15)
        def _z_last():
            pltpu.sync_copy(zer_h.at[pl.ds(0, RPT_LAST)],
                            acc.at[pl.ds(15 * RPT, RPT_LAST)])

        plsc.subcore_barrier()

        def issue_gather(b):
            cv, _, _, gv, _, _, gsem, _ = bufs[b]
            pltpu.async_copy(src_h.at[cv], gv, gsem)

        def wait_gather(b):
            cv, _, _, gv, _, _, gsem, _ = bufs[b]
            pltpu.make_async_copy(src_h.at[cv], gv, gsem).wait()

        def scale_scatter(b):
            _, rv, vv, gv, rs, _, _, ssem = bufs[b]

            @pl.loop(0, CHUNK // 16, unroll=5)
            def _sc(g):
                valv = vv[pl.ds(g * 16, 16)]
                for l in range(16):
                    e = g * 16 + l
                    vb = jnp.full((16,), valv[l], jnp.float32)
                    gv[e, pl.ds(0, 16)] = gv[e, pl.ds(0, 16)] * vb
                    gv[e, pl.ds(16, 16)] = gv[e, pl.ds(16, 16)] * vb
                rs[pl.ds(g * 16, 16)] = rv[pl.ds(g * 16, 16)]

            pltpu.async_copy(gv, acc.at[rs], ssem, add=True)

        def wait_scatter(b):
            _, _, _, gv, rs, _, _, ssem = bufs[b]
            pltpu.make_async_copy(gv, acc.at[rs], ssem).wait()

        issue_idx(0, 0)
        issue_idx(1, 1)

        @pl.loop(0, NCH - 1, step=2)
        def _pair(t):
            for b in (0, 1):
                j = t + b
                wait_idx(b)

                @pl.when(t > 0)
                def _ws():
                    wait_scatter(b)

                issue_gather(b)
                ob = 1 - b
                if b == 0:
                    @pl.when(t > 0)
                    def _prev():
                        wait_gather(ob)
                        scale_scatter(ob)
                        issue_idx(j + 1, ob)
                else:
                    wait_gather(ob)
                    scale_scatter(ob)
                    issue_idx(j + 1, ob)

        wait_idx(0)
        wait_scatter(0)
        issue_gather(0)
        wait_gather(1)
        scale_scatter(1)
        wait_gather(0)
        scale_scatter(0)
        wait_scatter(1)
        wait_scatter(0)

        plsc.subcore_barrier()

        @pl.when(s < 15)
        def _w_main():
            pltpu.sync_copy(acc.at[pl.ds(s * RPT, RPT)],
                            dst_h.at[pl.ds(rowoff + s * RPT, RPT)])

        @pl.when(s == 15)
        def _w_last():
            pltpu.sync_copy(acc.at[pl.ds(15 * RPT, RPT_LAST)],
                            dst_h.at[pl.ds(rowoff + 15 * RPT, RPT_LAST)])

        plsc.subcore_barrier()

    run_layer(th0_h, th1_h)
    run_layer(th1_h, th2_h)


def _propagate(graph_rows, graph_cols, graph_vals, th0, zer):
    f = pl.kernel(
        _prop_body,
        out_type=(jax.ShapeDtypeStruct((2 * N, CH), jnp.float32),
                  jax.ShapeDtypeStruct((2 * N, CH), jnp.float32)),
        mesh=_SC_MESH,
        scratch_types=(
            pltpu.VMEM_SHARED((N, CH), jnp.float32),
            pltpu.VMEM((CHUNK,), jnp.int32), pltpu.VMEM((CHUNK,), jnp.int32),
            pltpu.VMEM((CHUNK,), jnp.int32), pltpu.VMEM((CHUNK,), jnp.int32),
            pltpu.VMEM((CHUNK,), jnp.float32), pltpu.VMEM((CHUNK,), jnp.float32),
            pltpu.VMEM((CHUNK, CH), jnp.float32), pltpu.VMEM((CHUNK, CH), jnp.float32),
            pltpu.VMEM((CHUNK,), jnp.int32), pltpu.VMEM((CHUNK,), jnp.int32),
            pltpu.SemaphoreType.DMA, pltpu.SemaphoreType.DMA,
            pltpu.SemaphoreType.DMA, pltpu.SemaphoreType.DMA,
            pltpu.SemaphoreType.DMA, pltpu.SemaphoreType.DMA,
        ),
        compiler_params=pltpu.CompilerParams(use_tc_tiling_on_sc=False),
    )
    return f(graph_rows, graph_cols, graph_vals, th0, zer)


NW = NSC * NTILES
BPW = B // NW
BCHUNK = 16
NPB = 7
GR = BCHUNK * NPB


def _sigmoid(x):
    return 1.0 / (1.0 + jnp.exp(-x))


def _batch_body(idx7_h, w1_h, w2_h, th0_h, th1_h, th2_h,
                o_adj_h, o_int_h, o_dis_h, o_sadj_h, o_sint_h, o_sdis_h,
                idx_v, idxo_v, g0l, g0h, g1l, g1h, g2l, g2h, w1g, w2g,
                b_adj, b_int, b_dis, b_sadj, b_sint, b_sdis, gsem):
    c = lax.axis_index("c")
    s = lax.axis_index("s")
    w = s * NSC + c
    base_b = w * BPW

    obufs = (b_adj, b_int, b_dis, b_sadj, b_sint, b_sdis)

    @pl.loop(0, BPW // BCHUNK)
    def _chunk(t):
        ib = (base_b + t * BCHUNK) * NPB
        pltpu.sync_copy(idx7_h.at[pl.ds(ib, GR)], idx_v)
        for i in range(GR // 16):
            idxo_v[pl.ds(i * 16, 16)] = idx_v[pl.ds(i * 16, 16)] + N
        cps = (
            pltpu.async_copy(th0_h.at[idx_v], g0l, gsem),
            pltpu.async_copy(th0_h.at[idxo_v], g0h, gsem),
            pltpu.async_copy(th1_h.at[idx_v], g1l, gsem),
            pltpu.async_copy(th1_h.at[idxo_v], g1h, gsem),
            pltpu.async_copy(th2_h.at[idx_v], g2l, gsem),
            pltpu.async_copy(th2_h.at[idxo_v], g2h, gsem),
            pltpu.async_copy(w1_h.at[idx_v], w1g, gsem),
            pltpu.async_copy(w2_h.at[idx_v], w2g, gsem),
        )
        for cp in cps:
            cp.wait()

        @pl.loop(0, BCHUNK // 16)
        def _group(g):
            lanes = lax.iota(jnp.int32, 16)
            rowb = (g * 16 + lanes) * NPB
            rows = [rowb + p for p in range(NPB)]
            w1v = [plsc.load_gather(w1g, [rows[p]]) for p in range(NPB)]
            w2v = [plsc.load_gather(w2g, [rows[p]]) for p in range(NPB)]

            def half_accum(garrs, f, accs):
                colv = jnp.bitwise_and(jnp.full((16,), f, jnp.int32) + lanes,
                                       CH - 1)
                x0 = [plsc.load_gather(garrs[0], [rows[p], colv])
                      for p in range(NPB)]
                x1 = [plsc.load_gather(garrs[1], [rows[p], colv])
                      for p in range(NPB)]
                x2 = [plsc.load_gather(garrs[2], [rows[p], colv])
                      for p in range(NPB)]
                z1 = [_sigmoid(x1[p] * w1v[p] + w2v[p]) for p in range(NPB)]
                z2 = [_sigmoid(x2[p] * w1v[p] + w2v[p]) for p in range(NPB)]
                z0 = x0
                lgc = [(x0[p] + x1[p] + x2[p]) * (1.0 / 3.0)
                       for p in range(NPB)]
                lgc_mi = (lgc[2] + lgc[3]) * 0.5
                lgc_md = (lgc[4] + lgc[5] + lgc[6]) * (1.0 / 3.0)
                zmi = [(zk[2] + zk[3]) * 0.5 for zk in (z0, z1, z2)]
                zmd = [(zk[4] + zk[5] + zk[6]) * (1.0 / 3.0)
                       for zk in (z0, z1, z2)]
                a_adj, a_int, a_dis, a_sadj, a_sint, a_sdis = accs
                a_adj = a_adj + lgc[0] * lgc[1]
                a_int = a_int + lgc[0] * lgc_mi
                a_dis = a_dis + lgc[0] * lgc_md
                a_sadj = a_sadj + (z0[0] * z1[1] + z1[0] * z0[1]
                                   + z1[0] * z2[1] + z2[0] * z1[1])
                a_sint = a_sint + (z0[0] * zmi[1] + z1[0] * zmi[0]
                                   + z1[0] * zmi[2] + z2[0] * zmi[1])
                a_sdis = a_sdis + (z0[0] * zmd[1] + z1[0] * zmd[0]
                                   + z1[0] * zmd[2] + z2[0] * zmd[1])
                return (a_adj, a_int, a_dis, a_sadj, a_sint, a_sdis)

            zero = jnp.zeros((16,), jnp.float32)
            accs = (zero,) * 6
            accs = lax.fori_loop(
                0, CH, lambda f, a: half_accum((g0l, g1l, g2l), f, a), accs)
            accs = lax.fori_loop(
                0, CH, lambda f, a: half_accum((g0h, g1h, g2h), f, a), accs)
            a_adj, a_int, a_dis, a_sadj, a_sint, a_sdis = accs
            off = t * BCHUNK + g * 16
            b_adj[pl.ds(off, 16)] = _sigmoid(a_adj)
            b_int[pl.ds(off, 16)] = _sigmoid(a_int)
            b_dis[pl.ds(off, 16)] = _sigmoid(a_dis)
            b_sadj[pl.ds(off, 16)] = a_sadj * 0.25
            b_sint[pl.ds(off, 16)] = a_sint * 0.25
            b_sdis[pl.ds(off, 16)] = a_sdis * 0.25

    outs = (o_adj_h, o_int_h, o_dis_h, o_sadj_h, o_sint_h, o_sdis_h)
    for ob, oh in zip(obufs, outs):
        pltpu.sync_copy(ob, oh.at[pl.ds(base_b, BPW)])


def _batch_phase(idx7, w1, w2, th0, th1, th2):
    out1 = jax.ShapeDtypeStruct((B,), jnp.float32)
    f = pl.kernel(
        _batch_body,
        out_type=(out1,) * 6,
        mesh=_SC_MESH,
        scratch_types=(
            pltpu.VMEM((GR,), jnp.int32), pltpu.VMEM((GR,), jnp.int32),
            pltpu.VMEM((GR, CH), jnp.float32), pltpu.VMEM((GR, CH), jnp.float32),
            pltpu.VMEM((GR, CH), jnp.float32), pltpu.VMEM((GR, CH), jnp.float32),
            pltpu.VMEM((GR, CH), jnp.float32), pltpu.VMEM((GR, CH), jnp.float32),
            pltpu.VMEM((GR,), jnp.float32), pltpu.VMEM((GR,), jnp.float32),
            pltpu.VMEM((BPW,), jnp.float32), pltpu.VMEM((BPW,), jnp.float32),
            pltpu.VMEM((BPW,), jnp.float32), pltpu.VMEM((BPW,), jnp.float32),
            pltpu.VMEM((BPW,), jnp.float32), pltpu.VMEM((BPW,), jnp.float32),
            pltpu.SemaphoreType.DMA,
        ),
        compiler_params=pltpu.CompilerParams(
            use_tc_tiling_on_sc=False, needs_layout_passes=False),
    )
    return f(idx7, w1, w2, th0, th1, th2)


def kernel(users, adjacent_items, intermediate_items, distant_items,
           graph_rows, graph_cols, graph_vals,
           theta_user, theta_item, w1_user, w2_user, w1_item, w2_item):
    theta_cat = jnp.concatenate([theta_user, theta_item], axis=0)
    th0 = _softmax_stacked(theta_cat)

    zer = jnp.zeros((RPT, CH), jnp.float32)
    rows32 = graph_rows.astype(jnp.int32)
    cols32 = graph_cols.astype(jnp.int32)
    cols2 = jnp.concatenate([cols32, cols32 + N])
    th1, th2 = _propagate(rows32, cols2, graph_vals, th0, zer)

    items = jnp.concatenate(
        [adjacent_items[:, None], intermediate_items, distant_items], axis=1)
    idx7 = jnp.concatenate(
        [users[:, None], items + NUM_USERS], axis=1).astype(jnp.int32).reshape(-1)
    w1 = jnp.concatenate([w1_user[:, 0], w1_item[:, 0]])
    w2 = jnp.concatenate([w2_user[:, 0], w2_item[:, 0]])

    outs = _batch_phase(idx7, w1, w2, th0, th1, th2)
    return jnp.stack(outs, axis=0)
